# baseline (speedup 1.0000x reference)
import functools

import jax
import jax.numpy as jnp
from jax import lax
from jax.experimental import pallas as pl
from jax.experimental.pallas import tpu as pltpu
from jax.experimental.pallas import tpu_sc as plsc

N = 10000
E = 320000
D = 128
DE = 16
DH = 256
DO = 128

NC = 2
NS = 16
NW = NC * NS
EPW = E // NW
CH = 80
NT = EPW // CH


def _sc_mesh():
    return plsc.VectorSubcoreMesh(
        core_axis_name="c", subcore_axis_name="s", num_cores=NC, num_subcores=NS
    )


def _sc_gather(nf, row2d, col2d):
    @functools.partial(
        pl.kernel,
        out_type=[
            jax.ShapeDtypeStruct((E, D), jnp.float32),
            jax.ShapeDtypeStruct((E, D), jnp.float32),
        ],
        mesh=_sc_mesh(),
        scratch_types=[
            pltpu.VMEM((NT, CH), jnp.int32),
            pltpu.VMEM((NT, CH), jnp.int32),
            pltpu.VMEM((CH, D), jnp.float32),
            pltpu.VMEM((CH, D), jnp.float32),
            pltpu.SemaphoreType.DMA,
            pltpu.SemaphoreType.DMA,
        ],
    )
    def k(nf_hbm, row_hbm, col_hbm, xr_hbm, xc_hbm,
          ridx_v, cidx_v, xr_v, xc_v, sem0, sem1):
        wid = lax.axis_index("c") * NS + lax.axis_index("s")
        pltpu.sync_copy(row_hbm.at[wid], ridx_v)
        pltpu.sync_copy(col_hbm.at[wid], cidx_v)

        def body(t, carry):
            off = pl.multiple_of(wid * EPW + t * CH, 8)
            cp0 = pltpu.async_copy(nf_hbm.at[ridx_v.at[t]], xr_v, sem0)
            cp1 = pltpu.async_copy(nf_hbm.at[cidx_v.at[t]], xc_v, sem1)
            cp0.wait()
            cp1.wait()
            pltpu.sync_copy(xr_v, xr_hbm.at[pl.ds(off, CH)])
            pltpu.sync_copy(xc_v, xc_hbm.at[pl.ds(off, CH)])
            return carry

        lax.fori_loop(0, NT, body, 0)

    return k(nf, row2d, col2d)


def _sc_scatter(messages, col2d, zeros):
    @functools.partial(
        pl.kernel,
        out_type=jax.ShapeDtypeStruct((NC * N, D), jnp.float32),
        mesh=_sc_mesh(),
        scratch_types=[
            pltpu.VMEM((NT, CH), jnp.int32),
            pltpu.VMEM((CH, D), jnp.float32),
            pltpu.VMEM_SHARED((N, D), jnp.float32),
        ],
    )
    def k(msg_hbm, col_hbm, zero_hbm, out_hbm, cidx_v, msg_v, agg_sh):
        cid = lax.axis_index("c")
        sid = lax.axis_index("s")
        wid = cid * NS + sid
        rz = 632
        zoff = pl.multiple_of(jnp.where(sid == NS - 1, N - rz, sid * rz), 8)
        pltpu.sync_copy(zero_hbm.at[pl.ds(zoff, rz)],
                        agg_sh.at[pl.ds(zoff, rz)])
        plsc.subcore_barrier()

        pltpu.sync_copy(col_hbm.at[wid], cidx_v)

        def body(t, carry):
            off = pl.multiple_of(wid * EPW + t * CH, 8)
            pltpu.sync_copy(msg_hbm.at[pl.ds(off, CH)], msg_v)
            pltpu.sync_copy(msg_v, agg_sh.at[cidx_v.at[t]], add=True)
            return carry

        lax.fori_loop(0, NT, body, 0)
        plsc.subcore_barrier()
        pltpu.sync_copy(agg_sh.at[pl.ds(zoff, rz)],
                        out_hbm.at[pl.ds(pl.multiple_of(cid * N + zoff, 8), rz)])

    return k(messages, col2d, zeros)


def _tc_messages(xr, xc, ea, wx, we, wo):
    BE = 2000

    def body(xr_ref, xc_ref, ea_ref, wx_ref, we_ref, wo_ref, out_ref):
        p = (xr_ref[...] * xc_ref[...]).astype(jnp.bfloat16)
        z = jnp.dot(p, wx_ref[...], preferred_element_type=jnp.float32)
        g = jnp.dot(ea_ref[...], we_ref[...], preferred_element_type=jnp.float32)
        z = z * g
        z = z * (1.0 / (1.0 + jnp.exp(-z)))
        out_ref[...] = jnp.dot(z.astype(jnp.bfloat16), wo_ref[...],
                               preferred_element_type=jnp.float32)

    return pl.pallas_call(
        body,
        grid=(E // BE,),
        in_specs=[
            pl.BlockSpec((BE, D), lambda i: (i, 0)),
            pl.BlockSpec((BE, D), lambda i: (i, 0)),
            pl.BlockSpec((BE, DE), lambda i: (i, 0)),
            pl.BlockSpec((D, DH), lambda i: (0, 0)),
            pl.BlockSpec((DE, DH), lambda i: (0, 0)),
            pl.BlockSpec((DH, DO), lambda i: (0, 0)),
        ],
        out_specs=pl.BlockSpec((BE, DO), lambda i: (i, 0)),
        out_shape=jax.ShapeDtypeStruct((E, DO), jnp.float32),
    )(xr, xc, ea, wx, we, wo)


def _tc_update(x, agg2, wx, wm, wo):
    BN = 1000
    nblk = N // BN

    def body(x_ref, a0_ref, a1_ref, wx_ref, wm_ref, wo_ref, out_ref):
        a = a0_ref[...] + a1_ref[...]
        u = jnp.dot(x_ref[...], wx_ref[...], preferred_element_type=jnp.float32)
        u = u * jnp.dot(a, wm_ref[...], preferred_element_type=jnp.float32)
        u = u * (1.0 / (1.0 + jnp.exp(-u)))
        out_ref[...] = jnp.dot(u, wo_ref[...], preferred_element_type=jnp.float32)

    return pl.pallas_call(
        body,
        grid=(nblk,),
        in_specs=[
            pl.BlockSpec((BN, D), lambda i: (i, 0)),
            pl.BlockSpec((BN, D), lambda i: (i, 0)),
            pl.BlockSpec((BN, D), lambda i: (i + nblk, 0)),
            pl.BlockSpec((D, DH), lambda i: (0, 0)),
            pl.BlockSpec((DO, DH), lambda i: (0, 0)),
            pl.BlockSpec((DH, DO), lambda i: (0, 0)),
        ],
        out_specs=pl.BlockSpec((BN, DO), lambda i: (i, 0)),
        out_shape=jax.ShapeDtypeStruct((N, DO), jnp.float32),
    )(x, agg2, agg2, wx, wm, wo)


def kernel(node_features, pos, edge_index, edge_attr,
           W_msg_x, W_msg_e, W_msg_out, W_upd_x, W_upd_m, W_upd_out):
    del pos
    row2d = edge_index[0].astype(jnp.int32).reshape(NW, NT, CH)
    col2d = edge_index[1].astype(jnp.int32).reshape(NW, NT, CH)
    xr, xc = _sc_gather(node_features, row2d, col2d)
    messages = _tc_messages(xr, xc, edge_attr,
                            W_msg_x.astype(jnp.bfloat16), W_msg_e,
                            W_msg_out.astype(jnp.bfloat16))
    zeros = jnp.zeros((N, D), jnp.float32)
    agg2 = _sc_scatter(messages, col2d, zeros)
    return _tc_update(node_features, agg2, W_upd_x, W_upd_m, W_upd_out)


# speedup vs baseline: 1.1663x; 1.1663x over previous
import functools

import jax
import jax.numpy as jnp
from jax import lax
from jax.experimental import pallas as pl
from jax.experimental.pallas import tpu as pltpu
from jax.experimental.pallas import tpu_sc as plsc

N = 10000
E = 320000
D = 128
DE = 16
DH = 256
DO = 128

NC = 2
NS = 16
NW = NC * NS
EPW = E // NW
CH = 80
NT = EPW // CH


def _sc_mesh():
    return plsc.VectorSubcoreMesh(
        core_axis_name="c", subcore_axis_name="s", num_cores=NC, num_subcores=NS
    )


def _sc_gather_prod(nf, row2d, col2d):
    @functools.partial(
        pl.kernel,
        out_type=jax.ShapeDtypeStruct((E, D), jnp.float32),
        mesh=_sc_mesh(),
        scratch_types=[
            pltpu.VMEM((NT, CH), jnp.int32),
            pltpu.VMEM((NT, CH), jnp.int32),
            pltpu.VMEM((CH, D), jnp.float32),
            pltpu.VMEM((CH, D), jnp.float32),
            pltpu.VMEM((CH, D), jnp.float32),
            pltpu.VMEM((CH, D), jnp.float32),
            pltpu.VMEM((CH, D), jnp.float32),
            pltpu.VMEM((CH, D), jnp.float32),
            pltpu.SemaphoreType.DMA,
            pltpu.SemaphoreType.DMA,
            pltpu.SemaphoreType.DMA,
            pltpu.SemaphoreType.DMA,
        ],
    )
    def k(nf_hbm, row_hbm, col_hbm, p_hbm,
          ridx, cidx, xa, ya, xb, yb, pa, pb, gsA, gsB, wsA, wsB):
        wid = lax.axis_index("c") * NS + lax.axis_index("s")
        pltpu.sync_copy(row_hbm.at[wid], ridx)
        pltpu.sync_copy(col_hbm.at[wid], cidx)

        def gather(t, xv, yv, sem):
            pltpu.async_copy(nf_hbm.at[ridx.at[t]], xv, sem)
            pltpu.async_copy(nf_hbm.at[cidx.at[t]], yv, sem)

        def gwait(xv, yv, sem):
            pltpu.make_async_copy(nf_hbm.at[ridx.at[0]], xv, sem).wait()
            pltpu.make_async_copy(nf_hbm.at[cidx.at[0]], yv, sem).wait()

        def compute(xv, yv, pv):
            def edge(e, carry):
                for q in range(8):
                    pv[e, pl.ds(16 * q, 16)] = (
                        xv[e, pl.ds(16 * q, 16)] * yv[e, pl.ds(16 * q, 16)])
                return carry

            lax.fori_loop(0, CH, edge, 0)

        def wstart(t, pv, sem):
            off = pl.multiple_of(wid * EPW + t * CH, 8)
            pltpu.async_copy(pv, p_hbm.at[pl.ds(off, CH)], sem)

        def wwait(pv, sem):
            pltpu.make_async_copy(pv, p_hbm.at[pl.ds(0, CH)], sem).wait()

        gather(0, xa, ya, gsA)

        def body(t, carry):
            @pl.when(t > 0)
            def _():
                wwait(pa, wsA)

            gather(2 * t + 1, xb, yb, gsB)
            gwait(xa, ya, gsA)
            compute(xa, ya, pa)
            wstart(2 * t, pa, wsA)
            gather(2 * t + 2, xa, ya, gsA)

            @pl.when(t > 0)
            def _():
                wwait(pb, wsB)

            gwait(xb, yb, gsB)
            compute(xb, yb, pb)
            wstart(2 * t + 1, pb, wsB)
            return carry

        lax.fori_loop(0, (NT - 1) // 2, body, 0)
        wwait(pb, wsB)
        wwait(pa, wsA)
        gwait(xa, ya, gsA)
        compute(xa, ya, pa)
        wstart(NT - 1, pa, wsA)
        wwait(pa, wsA)

    return k(nf, row2d, col2d)


def _tc_messages(p, ea, wx, we, wo):
    BE = 2000

    def body(p_ref, ea_ref, wx_ref, we_ref, wo_ref, out_ref):
        z = jnp.dot(p_ref[...].astype(jnp.bfloat16), wx_ref[...],
                    preferred_element_type=jnp.float32)
        g = jnp.dot(ea_ref[...], we_ref[...],
                    preferred_element_type=jnp.float32)
        z = z * g
        z = z * (1.0 / (1.0 + jnp.exp(-z)))
        out_ref[...] = jnp.dot(z.astype(jnp.bfloat16), wo_ref[...],
                               preferred_element_type=jnp.float32)

    return pl.pallas_call(
        body,
        grid=(E // BE,),
        in_specs=[
            pl.BlockSpec((BE, D), lambda i: (i, 0)),
            pl.BlockSpec((BE, DE), lambda i: (i, 0)),
            pl.BlockSpec((D, DH), lambda i: (0, 0)),
            pl.BlockSpec((DE, DH), lambda i: (0, 0)),
            pl.BlockSpec((DH, DO), lambda i: (0, 0)),
        ],
        out_specs=pl.BlockSpec((BE, DO), lambda i: (i, 0)),
        out_shape=jax.ShapeDtypeStruct((E, DO), jnp.float32),
    )(p, ea, wx, we, wo)


def _sc_scatter(messages, col2d, zeros):
    @functools.partial(
        pl.kernel,
        out_type=jax.ShapeDtypeStruct((NC * N, D), jnp.float32),
        mesh=_sc_mesh(),
        scratch_types=[
            pltpu.VMEM((NT, CH), jnp.int32),
            pltpu.VMEM((CH, D), jnp.float32),
            pltpu.VMEM_SHARED((N, D), jnp.float32),
        ],
    )
    def k(msg_hbm, col_hbm, zero_hbm, out_hbm, cidx_v, msg_v, agg_sh):
        cid = lax.axis_index("c")
        sid = lax.axis_index("s")
        wid = cid * NS + sid
        rz = 632
        zoff = pl.multiple_of(jnp.where(sid == NS - 1, N - rz, sid * rz), 8)
        pltpu.sync_copy(zero_hbm.at[pl.ds(zoff, rz)],
                        agg_sh.at[pl.ds(zoff, rz)])
        plsc.subcore_barrier()

        pltpu.sync_copy(col_hbm.at[wid], cidx_v)

        def body(t, carry):
            off = pl.multiple_of(wid * EPW + t * CH, 8)
            pltpu.sync_copy(msg_hbm.at[pl.ds(off, CH)], msg_v)
            pltpu.sync_copy(msg_v, agg_sh.at[cidx_v.at[t]], add=True)
            return carry

        lax.fori_loop(0, NT, body, 0)
        plsc.subcore_barrier()
        pltpu.sync_copy(agg_sh.at[pl.ds(zoff, rz)],
                        out_hbm.at[pl.ds(pl.multiple_of(cid * N + zoff, 8), rz)])

    return k(messages, col2d, zeros)


def _tc_update(x, agg2, wx, wm, wo):
    BN = 1000
    nblk = N // BN

    def body(x_ref, a0_ref, a1_ref, wx_ref, wm_ref, wo_ref, out_ref):
        a = a0_ref[...] + a1_ref[...]
        u = jnp.dot(x_ref[...], wx_ref[...], preferred_element_type=jnp.float32)
        u = u * jnp.dot(a, wm_ref[...], preferred_element_type=jnp.float32)
        u = u * (1.0 / (1.0 + jnp.exp(-u)))
        out_ref[...] = jnp.dot(u, wo_ref[...], preferred_element_type=jnp.float32)

    return pl.pallas_call(
        body,
        grid=(nblk,),
        in_specs=[
            pl.BlockSpec((BN, D), lambda i: (i, 0)),
            pl.BlockSpec((BN, D), lambda i: (i, 0)),
            pl.BlockSpec((BN, D), lambda i: (i + nblk, 0)),
            pl.BlockSpec((D, DH), lambda i: (0, 0)),
            pl.BlockSpec((DO, DH), lambda i: (0, 0)),
            pl.BlockSpec((DH, DO), lambda i: (0, 0)),
        ],
        out_specs=pl.BlockSpec((BN, DO), lambda i: (i, 0)),
        out_shape=jax.ShapeDtypeStruct((N, DO), jnp.float32),
    )(x, agg2, agg2, wx, wm, wo)


def kernel(node_features, pos, edge_index, edge_attr,
           W_msg_x, W_msg_e, W_msg_out, W_upd_x, W_upd_m, W_upd_out):
    del pos
    row2d = edge_index[0].astype(jnp.int32).reshape(NW, NT, CH)
    col2d = edge_index[1].astype(jnp.int32).reshape(NW, NT, CH)

    p = _sc_gather_prod(node_features, row2d, col2d)
    messages = _tc_messages(p, edge_attr, W_msg_x.astype(jnp.bfloat16),
                            W_msg_e, W_msg_out.astype(jnp.bfloat16))
    zeros = jnp.zeros((N, D), jnp.float32)
    agg2 = _sc_scatter(messages, col2d, zeros)
    return _tc_update(node_features, agg2, W_upd_x, W_upd_m, W_upd_out)


# speedup vs baseline: 1.3078x; 1.1214x over previous
import functools

import jax
import jax.numpy as jnp
from jax import lax
from jax.experimental import pallas as pl
from jax.experimental.pallas import tpu as pltpu
from jax.experimental.pallas import tpu_sc as plsc

N = 10000
E = 320000
D = 128
DE = 16
DH = 256
DO = 128

NC = 2
NS = 16
NW = NC * NS
EPW = E // NW
CH = 80
NT = EPW // CH


def _sc_mesh():
    return plsc.VectorSubcoreMesh(
        core_axis_name="c", subcore_axis_name="s", num_cores=NC, num_subcores=NS
    )


def _sc_gather_prod(nf, row2d, col2d):
    @functools.partial(
        pl.kernel,
        out_type=jax.ShapeDtypeStruct((E, D), jnp.float32),
        mesh=_sc_mesh(),
        scratch_types=[
            pltpu.VMEM((NT, CH), jnp.int32),
            pltpu.VMEM((NT, CH), jnp.int32),
            pltpu.VMEM((CH, D), jnp.float32),
            pltpu.VMEM((CH, D), jnp.float32),
            pltpu.VMEM((CH, D), jnp.float32),
            pltpu.VMEM((CH, D), jnp.float32),
            pltpu.VMEM((CH, D), jnp.float32),
            pltpu.VMEM((CH, D), jnp.float32),
            pltpu.SemaphoreType.DMA,
            pltpu.SemaphoreType.DMA,
            pltpu.SemaphoreType.DMA,
            pltpu.SemaphoreType.DMA,
        ],
    )
    def k(nf_hbm, row_hbm, col_hbm, p_hbm,
          ridx, cidx, xa, ya, xb, yb, pa, pb, gsA, gsB, wsA, wsB):
        wid = lax.axis_index("c") * NS + lax.axis_index("s")
        pltpu.sync_copy(row_hbm.at[wid], ridx)
        pltpu.sync_copy(col_hbm.at[wid], cidx)

        def gather(t, xv, yv, sem):
            pltpu.async_copy(nf_hbm.at[ridx.at[t]], xv, sem)
            pltpu.async_copy(nf_hbm.at[cidx.at[t]], yv, sem)

        def gwait(xv, yv, sem):
            pltpu.make_async_copy(nf_hbm.at[ridx.at[0]], xv, sem).wait()
            pltpu.make_async_copy(nf_hbm.at[cidx.at[0]], yv, sem).wait()

        def compute(xv, yv, pv):
            def edge(e, carry):
                for q in range(8):
                    pv[e, pl.ds(16 * q, 16)] = (
                        xv[e, pl.ds(16 * q, 16)] * yv[e, pl.ds(16 * q, 16)])
                return carry

            lax.fori_loop(0, CH, edge, 0)

        def wstart(t, pv, sem):
            off = pl.multiple_of(wid * EPW + t * CH, 8)
            pltpu.async_copy(pv, p_hbm.at[pl.ds(off, CH)], sem)

        def wwait(pv, sem):
            pltpu.make_async_copy(pv, p_hbm.at[pl.ds(0, CH)], sem).wait()

        gather(0, xa, ya, gsA)

        def body(t, carry):
            @pl.when(t > 0)
            def _():
                wwait(pa, wsA)

            gather(2 * t + 1, xb, yb, gsB)
            gwait(xa, ya, gsA)
            compute(xa, ya, pa)
            wstart(2 * t, pa, wsA)
            gather(2 * t + 2, xa, ya, gsA)

            @pl.when(t > 0)
            def _():
                wwait(pb, wsB)

            gwait(xb, yb, gsB)
            compute(xb, yb, pb)
            wstart(2 * t + 1, pb, wsB)
            return carry

        lax.fori_loop(0, (NT - 1) // 2, body, 0)
        wwait(pb, wsB)
        wwait(pa, wsA)
        gwait(xa, ya, gsA)
        compute(xa, ya, pa)
        wstart(NT - 1, pa, wsA)
        wwait(pa, wsA)

    return k(nf, row2d, col2d)


def _tc_messages(p, ea, wx, we, wo):
    BE = 2000

    def body(p_ref, ea_ref, wx_ref, we_ref, wo_ref, out_ref):
        z = jnp.dot(p_ref[...].astype(jnp.bfloat16), wx_ref[...],
                    preferred_element_type=jnp.float32)
        g = jnp.dot(ea_ref[...], we_ref[...],
                    preferred_element_type=jnp.float32)
        z = z * g
        z = z * (1.0 / (1.0 + jnp.exp(-z)))
        out_ref[...] = jnp.dot(z.astype(jnp.bfloat16), wo_ref[...],
                               preferred_element_type=jnp.float32)

    return pl.pallas_call(
        body,
        grid=(E // BE,),
        in_specs=[
            pl.BlockSpec((BE, D), lambda i: (i, 0)),
            pl.BlockSpec((BE, DE), lambda i: (i, 0)),
            pl.BlockSpec((D, DH), lambda i: (0, 0)),
            pl.BlockSpec((DE, DH), lambda i: (0, 0)),
            pl.BlockSpec((DH, DO), lambda i: (0, 0)),
        ],
        out_specs=pl.BlockSpec((BE, DO), lambda i: (i, 0)),
        out_shape=jax.ShapeDtypeStruct((E, DO), jnp.float32),
    )(p, ea, wx, we, wo)


def _sc_scatter(messages, col2d, zeros):
    @functools.partial(
        pl.kernel,
        out_type=jax.ShapeDtypeStruct((NC * N, D), jnp.float32),
        mesh=_sc_mesh(),
        scratch_types=[
            pltpu.VMEM((NT, CH), jnp.int32),
            pltpu.VMEM((CH, D), jnp.float32),
            pltpu.VMEM((CH, D), jnp.float32),
            pltpu.VMEM_SHARED((N, D), jnp.float32),
            pltpu.SemaphoreType.DMA,
            pltpu.SemaphoreType.DMA,
        ],
    )
    def k(msg_hbm, col_hbm, zero_hbm, out_hbm,
          cidx_v, ma, mb, agg_sh, lsA, lsB):
        cid = lax.axis_index("c")
        sid = lax.axis_index("s")
        wid = cid * NS + sid
        rz = 632
        zoff = pl.multiple_of(jnp.where(sid == NS - 1, N - rz, sid * rz), 8)
        pltpu.sync_copy(zero_hbm.at[pl.ds(zoff, rz)],
                        agg_sh.at[pl.ds(zoff, rz)])
        plsc.subcore_barrier()

        pltpu.sync_copy(col_hbm.at[wid], cidx_v)

        def load(t, mv, sem):
            off = pl.multiple_of(wid * EPW + t * CH, 8)
            pltpu.async_copy(msg_hbm.at[pl.ds(off, CH)], mv, sem)

        def lwait(mv, sem):
            pltpu.make_async_copy(msg_hbm.at[pl.ds(0, CH)], mv, sem).wait()

        def scat(t, mv):
            pltpu.sync_copy(mv, agg_sh.at[cidx_v.at[t]], add=True)

        load(0, ma, lsA)

        def body(t, carry):
            lwait(ma, lsA)
            load(2 * t + 1, mb, lsB)
            scat(2 * t, ma)
            load(2 * t + 2, ma, lsA)
            lwait(mb, lsB)
            scat(2 * t + 1, mb)
            return carry

        lax.fori_loop(0, (NT - 1) // 2, body, 0)
        lwait(ma, lsA)
        scat(NT - 1, ma)
        plsc.subcore_barrier()
        pltpu.sync_copy(agg_sh.at[pl.ds(zoff, rz)],
                        out_hbm.at[pl.ds(pl.multiple_of(cid * N + zoff, 8), rz)])

    return k(messages, col2d, zeros)


def _tc_update(x, agg2, wx, wm, wo):
    BN = 1000
    nblk = N // BN

    def body(x_ref, a0_ref, a1_ref, wx_ref, wm_ref, wo_ref, out_ref):
        a = a0_ref[...] + a1_ref[...]
        u = jnp.dot(x_ref[...], wx_ref[...], preferred_element_type=jnp.float32)
        u = u * jnp.dot(a, wm_ref[...], preferred_element_type=jnp.float32)
        u = u * (1.0 / (1.0 + jnp.exp(-u)))
        out_ref[...] = jnp.dot(u, wo_ref[...], preferred_element_type=jnp.float32)

    return pl.pallas_call(
        body,
        grid=(nblk,),
        in_specs=[
            pl.BlockSpec((BN, D), lambda i: (i, 0)),
            pl.BlockSpec((BN, D), lambda i: (i, 0)),
            pl.BlockSpec((BN, D), lambda i: (i + nblk, 0)),
            pl.BlockSpec((D, DH), lambda i: (0, 0)),
            pl.BlockSpec((DO, DH), lambda i: (0, 0)),
            pl.BlockSpec((DH, DO), lambda i: (0, 0)),
        ],
        out_specs=pl.BlockSpec((BN, DO), lambda i: (i, 0)),
        out_shape=jax.ShapeDtypeStruct((N, DO), jnp.float32),
    )(x, agg2, agg2, wx, wm, wo)


def kernel(node_features, pos, edge_index, edge_attr,
           W_msg_x, W_msg_e, W_msg_out, W_upd_x, W_upd_m, W_upd_out):
    del pos
    row2d = edge_index[0].astype(jnp.int32).reshape(NW, NT, CH)
    col2d = edge_index[1].astype(jnp.int32).reshape(NW, NT, CH)

    p = _sc_gather_prod(node_features, row2d, col2d)
    messages = _tc_messages(p, edge_attr, W_msg_x.astype(jnp.bfloat16),
                            W_msg_e, W_msg_out.astype(jnp.bfloat16))
    zeros = jnp.zeros((N, D), jnp.float32)
    agg2 = _sc_scatter(messages, col2d, zeros)
    return _tc_update(node_features, agg2, W_upd_x, W_upd_m, W_upd_out)


# speedup vs baseline: 1.3563x; 1.0371x over previous
import functools

import jax
import jax.numpy as jnp
from jax import lax
from jax.experimental import pallas as pl
from jax.experimental.pallas import tpu as pltpu
from jax.experimental.pallas import tpu_sc as plsc

N = 10000
E = 320000
D = 128
DE = 16
DH = 256
DO = 128

NC = 2
NS = 16
NW = NC * NS
NHALF = 2
EH = E // NHALF
EPW = EH // NW
CH = 40
NT = EPW // CH


def _sc_mesh():
    return plsc.VectorSubcoreMesh(
        core_axis_name="c", subcore_axis_name="s", num_cores=NC, num_subcores=NS
    )


def _sc_gather_prod(nf, row2d, col2d):
    @functools.partial(
        pl.kernel,
        out_type=jax.ShapeDtypeStruct((EH, D), jnp.float32),
        mesh=_sc_mesh(),
        scratch_types=[
            pltpu.VMEM((NT, CH), jnp.int32),
            pltpu.VMEM((NT, CH), jnp.int32),
            pltpu.VMEM((CH, D), jnp.float32),
            pltpu.VMEM((CH, D), jnp.float32),
            pltpu.VMEM((CH, D), jnp.float32),
            pltpu.VMEM((CH, D), jnp.float32),
            pltpu.VMEM((CH, D), jnp.float32),
            pltpu.VMEM((CH, D), jnp.float32),
            pltpu.SemaphoreType.DMA,
            pltpu.SemaphoreType.DMA,
            pltpu.SemaphoreType.DMA,
            pltpu.SemaphoreType.DMA,
        ],
    )
    def k(nf_hbm, row_hbm, col_hbm, p_hbm,
          ridx, cidx, xa, ya, xb, yb, pa, pb, gsA, gsB, wsA, wsB):
        wid = lax.axis_index("c") * NS + lax.axis_index("s")
        pltpu.sync_copy(row_hbm.at[wid], ridx)
        pltpu.sync_copy(col_hbm.at[wid], cidx)

        def gather(t, xv, yv, sem):
            pltpu.async_copy(nf_hbm.at[ridx.at[t]], xv, sem)
            pltpu.async_copy(nf_hbm.at[cidx.at[t]], yv, sem)

        def gwait(xv, yv, sem):
            pltpu.make_async_copy(nf_hbm.at[ridx.at[0]], xv, sem).wait()
            pltpu.make_async_copy(nf_hbm.at[cidx.at[0]], yv, sem).wait()

        def compute(xv, yv, pv):
            def edge(e, carry):
                for q in range(8):
                    pv[e, pl.ds(16 * q, 16)] = (
                        xv[e, pl.ds(16 * q, 16)] * yv[e, pl.ds(16 * q, 16)])
                return carry

            lax.fori_loop(0, CH, edge, 0)

        def wstart(t, pv, sem):
            off = pl.multiple_of(wid * EPW + t * CH, 8)
            pltpu.async_copy(pv, p_hbm.at[pl.ds(off, CH)], sem)

        def wwait(pv, sem):
            pltpu.make_async_copy(pv, p_hbm.at[pl.ds(0, CH)], sem).wait()

        gather(0, xa, ya, gsA)

        def body(t, carry):
            @pl.when(t > 0)
            def _():
                wwait(pa, wsA)

            gather(2 * t + 1, xb, yb, gsB)
            gwait(xa, ya, gsA)
            compute(xa, ya, pa)
            wstart(2 * t, pa, wsA)
            gather(2 * t + 2, xa, ya, gsA)

            @pl.when(t > 0)
            def _():
                wwait(pb, wsB)

            gwait(xb, yb, gsB)
            compute(xb, yb, pb)
            wstart(2 * t + 1, pb, wsB)
            return carry

        lax.fori_loop(0, (NT - 1) // 2, body, 0)
        wwait(pb, wsB)
        wwait(pa, wsA)
        gwait(xa, ya, gsA)
        compute(xa, ya, pa)
        wstart(NT - 1, pa, wsA)
        wwait(pa, wsA)

    return k(nf, row2d, col2d)


def _tc_messages(p, ea, wx, we, wo):
    BE = 2000

    def body(p_ref, ea_ref, wx_ref, we_ref, wo_ref, out_ref):
        z = jnp.dot(p_ref[...].astype(jnp.bfloat16), wx_ref[...],
                    preferred_element_type=jnp.float32)
        g = jnp.dot(ea_ref[...], we_ref[...],
                    preferred_element_type=jnp.float32)
        z = z * g
        z = z * (1.0 / (1.0 + jnp.exp(-z)))
        out_ref[...] = jnp.dot(z.astype(jnp.bfloat16), wo_ref[...],
                               preferred_element_type=jnp.float32)

    return pl.pallas_call(
        body,
        grid=(EH // BE,),
        in_specs=[
            pl.BlockSpec((BE, D), lambda i: (i, 0)),
            pl.BlockSpec((BE, DE), lambda i: (i, 0)),
            pl.BlockSpec((D, DH), lambda i: (0, 0)),
            pl.BlockSpec((DE, DH), lambda i: (0, 0)),
            pl.BlockSpec((DH, DO), lambda i: (0, 0)),
        ],
        out_specs=pl.BlockSpec((BE, DO), lambda i: (i, 0)),
        out_shape=jax.ShapeDtypeStruct((EH, DO), jnp.float32),
    )(p, ea, wx, we, wo)


def _sc_scatter(messages, col2d, zeros):
    @functools.partial(
        pl.kernel,
        out_type=jax.ShapeDtypeStruct((NC * N, D), jnp.float32),
        mesh=_sc_mesh(),
        scratch_types=[
            pltpu.VMEM((NT, CH), jnp.int32),
            pltpu.VMEM((CH, D), jnp.float32),
            pltpu.VMEM((CH, D), jnp.float32),
            pltpu.VMEM_SHARED((N, D), jnp.float32),
            pltpu.SemaphoreType.DMA,
            pltpu.SemaphoreType.DMA,
        ],
    )
    def k(msg_hbm, col_hbm, zero_hbm, out_hbm,
          cidx_v, ma, mb, agg_sh, lsA, lsB):
        cid = lax.axis_index("c")
        sid = lax.axis_index("s")
        wid = cid * NS + sid
        rz = 632
        zoff = pl.multiple_of(jnp.where(sid == NS - 1, N - rz, sid * rz), 8)
        pltpu.sync_copy(zero_hbm.at[pl.ds(zoff, rz)],
                        agg_sh.at[pl.ds(zoff, rz)])
        plsc.subcore_barrier()

        pltpu.sync_copy(col_hbm.at[wid], cidx_v)

        def load(t, mv, sem):
            off = pl.multiple_of(wid * EPW + t * CH, 8)
            pltpu.async_copy(msg_hbm.at[pl.ds(off, CH)], mv, sem)

        def lwait(mv, sem):
            pltpu.make_async_copy(msg_hbm.at[pl.ds(0, CH)], mv, sem).wait()

        def scat(t, mv):
            pltpu.sync_copy(mv, agg_sh.at[cidx_v.at[t]], add=True)

        load(0, ma, lsA)

        def body(t, carry):
            lwait(ma, lsA)
            load(2 * t + 1, mb, lsB)
            scat(2 * t, ma)
            load(2 * t + 2, ma, lsA)
            lwait(mb, lsB)
            scat(2 * t + 1, mb)
            return carry

        lax.fori_loop(0, (NT - 1) // 2, body, 0)
        lwait(ma, lsA)
        scat(NT - 1, ma)
        plsc.subcore_barrier()
        pltpu.sync_copy(agg_sh.at[pl.ds(zoff, rz)],
                        out_hbm.at[pl.ds(pl.multiple_of(cid * N + zoff, 8), rz)])

    return k(messages, col2d, zeros)


def _tc_update(x, agg_a, agg_b, wx, wm, wo):
    BN = 1000
    nblk = N // BN

    def body(x_ref, a0_ref, a1_ref, a2_ref, a3_ref,
             wx_ref, wm_ref, wo_ref, out_ref):
        a = (a0_ref[...] + a1_ref[...]) + (a2_ref[...] + a3_ref[...])
        u = jnp.dot(x_ref[...], wx_ref[...], preferred_element_type=jnp.float32)
        u = u * jnp.dot(a, wm_ref[...], preferred_element_type=jnp.float32)
        u = u * (1.0 / (1.0 + jnp.exp(-u)))
        out_ref[...] = jnp.dot(u, wo_ref[...], preferred_element_type=jnp.float32)

    return pl.pallas_call(
        body,
        grid=(nblk,),
        in_specs=[
            pl.BlockSpec((BN, D), lambda i: (i, 0)),
            pl.BlockSpec((BN, D), lambda i: (i, 0)),
            pl.BlockSpec((BN, D), lambda i: (i + nblk, 0)),
            pl.BlockSpec((BN, D), lambda i: (i, 0)),
            pl.BlockSpec((BN, D), lambda i: (i + nblk, 0)),
            pl.BlockSpec((D, DH), lambda i: (0, 0)),
            pl.BlockSpec((DO, DH), lambda i: (0, 0)),
            pl.BlockSpec((DH, DO), lambda i: (0, 0)),
        ],
        out_specs=pl.BlockSpec((BN, DO), lambda i: (i, 0)),
        out_shape=jax.ShapeDtypeStruct((N, DO), jnp.float32),
    )(x, agg_a, agg_a, agg_b, agg_b, wx, wm, wo)


def kernel(node_features, pos, edge_index, edge_attr,
           W_msg_x, W_msg_e, W_msg_out, W_upd_x, W_upd_m, W_upd_out):
    del pos
    row = edge_index[0].astype(jnp.int32)
    col = edge_index[1].astype(jnp.int32)
    wx16 = W_msg_x.astype(jnp.bfloat16)
    wo16 = W_msg_out.astype(jnp.bfloat16)
    zeros = jnp.zeros((N, D), jnp.float32)

    aggs = []
    for h in range(NHALF):
        sl = slice(h * EH, (h + 1) * EH)
        row2d = row[sl].reshape(NW, NT, CH)
        col2d = col[sl].reshape(NW, NT, CH)
        p = _sc_gather_prod(node_features, row2d, col2d)
        messages = _tc_messages(p, edge_attr[sl], wx16, W_msg_e, wo16)
        aggs.append(_sc_scatter(messages, col2d, zeros))

    return _tc_update(node_features, aggs[0], aggs[1],
                      W_upd_x, W_upd_m, W_upd_out)


# speedup vs baseline: 1.3686x; 1.0091x over previous
import functools

import jax
import jax.numpy as jnp
from jax import lax
from jax.experimental import pallas as pl
from jax.experimental.pallas import tpu as pltpu
from jax.experimental.pallas import tpu_sc as plsc

N = 10000
E = 320000
D = 128
DE = 16
DH = 256
DO = 128

NC = 2
NS = 16
NW = NC * NS
E0 = 128000
E1 = E - E0
CH0 = 32
CH1 = 48
NT = 125


def _sc_mesh():
    return plsc.VectorSubcoreMesh(
        core_axis_name="c", subcore_axis_name="s", num_cores=NC, num_subcores=NS
    )


def _sc_gather_prod(nf, row2d, col2d, eh, ch):
    epw = eh // NW
    CH = ch

    @functools.partial(
        pl.kernel,
        out_type=jax.ShapeDtypeStruct((eh, D), jnp.float32),
        mesh=_sc_mesh(),
        scratch_types=[
            pltpu.VMEM((NT, CH), jnp.int32),
            pltpu.VMEM((NT, CH), jnp.int32),
            pltpu.VMEM((CH, D), jnp.float32),
            pltpu.VMEM((CH, D), jnp.float32),
            pltpu.VMEM((CH, D), jnp.float32),
            pltpu.VMEM((CH, D), jnp.float32),
            pltpu.VMEM((CH, D), jnp.float32),
            pltpu.VMEM((CH, D), jnp.float32),
            pltpu.SemaphoreType.DMA,
            pltpu.SemaphoreType.DMA,
            pltpu.SemaphoreType.DMA,
            pltpu.SemaphoreType.DMA,
        ],
    )
    def k(nf_hbm, row_hbm, col_hbm, p_hbm,
          ridx, cidx, xa, ya, xb, yb, pa, pb, gsA, gsB, wsA, wsB):
        wid = lax.axis_index("c") * NS + lax.axis_index("s")
        pltpu.sync_copy(row_hbm.at[wid], ridx)
        pltpu.sync_copy(col_hbm.at[wid], cidx)

        def gather(t, xv, yv, sem):
            pltpu.async_copy(nf_hbm.at[ridx.at[t]], xv, sem)
            pltpu.async_copy(nf_hbm.at[cidx.at[t]], yv, sem)

        def gwait(xv, yv, sem):
            pltpu.make_async_copy(nf_hbm.at[ridx.at[0]], xv, sem).wait()
            pltpu.make_async_copy(nf_hbm.at[cidx.at[0]], yv, sem).wait()

        def compute(xv, yv, pv):
            def edge(e, carry):
                for q in range(8):
                    pv[e, pl.ds(16 * q, 16)] = (
                        xv[e, pl.ds(16 * q, 16)] * yv[e, pl.ds(16 * q, 16)])
                return carry

            lax.fori_loop(0, CH, edge, 0)

        def wstart(t, pv, sem):
            off = pl.multiple_of(wid * epw + t * CH, 8)
            pltpu.async_copy(pv, p_hbm.at[pl.ds(off, CH)], sem)

        def wwait(pv, sem):
            pltpu.make_async_copy(pv, p_hbm.at[pl.ds(0, CH)], sem).wait()

        gather(0, xa, ya, gsA)

        def body(t, carry):
            @pl.when(t > 0)
            def _():
                wwait(pa, wsA)

            gather(2 * t + 1, xb, yb, gsB)
            gwait(xa, ya, gsA)
            compute(xa, ya, pa)
            wstart(2 * t, pa, wsA)
            gather(2 * t + 2, xa, ya, gsA)

            @pl.when(t > 0)
            def _():
                wwait(pb, wsB)

            gwait(xb, yb, gsB)
            compute(xb, yb, pb)
            wstart(2 * t + 1, pb, wsB)
            return carry

        lax.fori_loop(0, (NT - 1) // 2, body, 0)
        wwait(pb, wsB)
        wwait(pa, wsA)
        gwait(xa, ya, gsA)
        compute(xa, ya, pa)
        wstart(NT - 1, pa, wsA)
        wwait(pa, wsA)

    return k(nf, row2d, col2d)


def _tc_messages(p, ea, wx, we, wo, eh):
    BE = 2000

    def body(p_ref, ea_ref, wx_ref, we_ref, wo_ref, out_ref):
        z = jnp.dot(p_ref[...].astype(jnp.bfloat16), wx_ref[...],
                    preferred_element_type=jnp.float32)
        g = jnp.dot(ea_ref[...], we_ref[...],
                    preferred_element_type=jnp.float32)
        z = z * g
        z = z * (1.0 / (1.0 + jnp.exp(-z)))
        out_ref[...] = jnp.dot(z.astype(jnp.bfloat16), wo_ref[...],
                               preferred_element_type=jnp.float32)

    return pl.pallas_call(
        body,
        grid=(eh // BE,),
        in_specs=[
            pl.BlockSpec((BE, D), lambda i: (i, 0)),
            pl.BlockSpec((BE, DE), lambda i: (i, 0)),
            pl.BlockSpec((D, DH), lambda i: (0, 0)),
            pl.BlockSpec((DE, DH), lambda i: (0, 0)),
            pl.BlockSpec((DH, DO), lambda i: (0, 0)),
        ],
        out_specs=pl.BlockSpec((BE, DO), lambda i: (i, 0)),
        out_shape=jax.ShapeDtypeStruct((eh, DO), jnp.float32),
    )(p, ea, wx, we, wo)


def _sc_scatter(messages, col2d, zeros, eh, ch):
    epw = eh // NW
    CH = ch

    @functools.partial(
        pl.kernel,
        out_type=jax.ShapeDtypeStruct((NC * N, D), jnp.float32),
        mesh=_sc_mesh(),
        scratch_types=[
            pltpu.VMEM((NT, CH), jnp.int32),
            pltpu.VMEM((CH, D), jnp.float32),
            pltpu.VMEM((CH, D), jnp.float32),
            pltpu.VMEM_SHARED((N, D), jnp.float32),
            pltpu.SemaphoreType.DMA,
            pltpu.SemaphoreType.DMA,
        ],
    )
    def k(msg_hbm, col_hbm, zero_hbm, out_hbm,
          cidx_v, ma, mb, agg_sh, lsA, lsB):
        cid = lax.axis_index("c")
        sid = lax.axis_index("s")
        wid = cid * NS + sid
        rz = 632
        zoff = pl.multiple_of(jnp.where(sid == NS - 1, N - rz, sid * rz), 8)
        pltpu.sync_copy(zero_hbm.at[pl.ds(zoff, rz)],
                        agg_sh.at[pl.ds(zoff, rz)])
        plsc.subcore_barrier()

        pltpu.sync_copy(col_hbm.at[wid], cidx_v)

        def load(t, mv, sem):
            off = pl.multiple_of(wid * epw + t * CH, 8)
            pltpu.async_copy(msg_hbm.at[pl.ds(off, CH)], mv, sem)

        def lwait(mv, sem):
            pltpu.make_async_copy(msg_hbm.at[pl.ds(0, CH)], mv, sem).wait()

        def scat(t, mv):
            pltpu.sync_copy(mv, agg_sh.at[cidx_v.at[t]], add=True)

        load(0, ma, lsA)

        def body(t, carry):
            lwait(ma, lsA)
            load(2 * t + 1, mb, lsB)
            scat(2 * t, ma)
            load(2 * t + 2, ma, lsA)
            lwait(mb, lsB)
            scat(2 * t + 1, mb)
            return carry

        lax.fori_loop(0, (NT - 1) // 2, body, 0)
        lwait(ma, lsA)
        scat(NT - 1, ma)
        plsc.subcore_barrier()
        pltpu.sync_copy(agg_sh.at[pl.ds(zoff, rz)],
                        out_hbm.at[pl.ds(pl.multiple_of(cid * N + zoff, 8), rz)])

    return k(messages, col2d, zeros)


def _tc_update(x, agg_a, agg_b, wx, wm, wo):
    BN = 1000
    nblk = N // BN

    def body(x_ref, a0_ref, a1_ref, a2_ref, a3_ref,
             wx_ref, wm_ref, wo_ref, out_ref):
        a = (a0_ref[...] + a1_ref[...]) + (a2_ref[...] + a3_ref[...])
        u = jnp.dot(x_ref[...], wx_ref[...], preferred_element_type=jnp.float32)
        u = u * jnp.dot(a, wm_ref[...], preferred_element_type=jnp.float32)
        u = u * (1.0 / (1.0 + jnp.exp(-u)))
        out_ref[...] = jnp.dot(u, wo_ref[...], preferred_element_type=jnp.float32)

    return pl.pallas_call(
        body,
        grid=(nblk,),
        in_specs=[
            pl.BlockSpec((BN, D), lambda i: (i, 0)),
            pl.BlockSpec((BN, D), lambda i: (i, 0)),
            pl.BlockSpec((BN, D), lambda i: (i + nblk, 0)),
            pl.BlockSpec((BN, D), lambda i: (i, 0)),
            pl.BlockSpec((BN, D), lambda i: (i + nblk, 0)),
            pl.BlockSpec((D, DH), lambda i: (0, 0)),
            pl.BlockSpec((DO, DH), lambda i: (0, 0)),
            pl.BlockSpec((DH, DO), lambda i: (0, 0)),
        ],
        out_specs=pl.BlockSpec((BN, DO), lambda i: (i, 0)),
        out_shape=jax.ShapeDtypeStruct((N, DO), jnp.float32),
    )(x, agg_a, agg_a, agg_b, agg_b, wx, wm, wo)


def kernel(node_features, pos, edge_index, edge_attr,
           W_msg_x, W_msg_e, W_msg_out, W_upd_x, W_upd_m, W_upd_out):
    del pos
    row = edge_index[0].astype(jnp.int32)
    col = edge_index[1].astype(jnp.int32)
    wx16 = W_msg_x.astype(jnp.bfloat16)
    wo16 = W_msg_out.astype(jnp.bfloat16)
    zeros = jnp.zeros((N, D), jnp.float32)

    aggs = []
    for h, (e0, eh, ch) in enumerate([(0, E0, CH0), (E0, E1, CH1)]):
        sl = slice(e0, e0 + eh)
        row2d = row[sl].reshape(NW, NT, ch)
        col2d = col[sl].reshape(NW, NT, ch)
        p = _sc_gather_prod(node_features, row2d, col2d, eh, ch)
        messages = _tc_messages(p, edge_attr[sl], wx16, W_msg_e, wo16, eh)
        aggs.append(_sc_scatter(messages, col2d, zeros, eh, ch))

    return _tc_update(node_features, aggs[0], aggs[1],
                      W_upd_x, W_upd_m, W_upd_out)


# speedup vs baseline: 1.4955x; 1.0928x over previous
import functools

import jax
import jax.numpy as jnp
from jax import lax
from jax.experimental import pallas as pl
from jax.experimental.pallas import tpu as pltpu
from jax.experimental.pallas import tpu_sc as plsc

N = 10000
E = 320000
D = 128
DE = 16
DH = 256
DO = 128

NC = 2
NS = 16
NW = NC * NS
E0 = 128000
E1 = E - E0
CH = 80
BE = 2000


def _sc_mesh():
    return plsc.VectorSubcoreMesh(
        core_axis_name="c", subcore_axis_name="s", num_cores=NC, num_subcores=NS
    )


def _sc_gather_prod(nf, row2d, col2d, eh):
    epw = eh // NW
    NT = epw // CH

    @functools.partial(
        pl.kernel,
        out_type=jax.ShapeDtypeStruct((eh, D), jnp.float32),
        mesh=_sc_mesh(),
        scratch_types=[
            pltpu.VMEM((NT, CH), jnp.int32),
            pltpu.VMEM((NT, CH), jnp.int32),
            pltpu.VMEM((CH, D), jnp.float32),
            pltpu.VMEM((CH, D), jnp.float32),
            pltpu.VMEM((CH, D), jnp.float32),
            pltpu.VMEM((CH, D), jnp.float32),
            pltpu.VMEM((CH, D), jnp.float32),
            pltpu.VMEM((CH, D), jnp.float32),
            pltpu.SemaphoreType.DMA,
            pltpu.SemaphoreType.DMA,
            pltpu.SemaphoreType.DMA,
            pltpu.SemaphoreType.DMA,
        ],
    )
    def k(nf_hbm, row_hbm, col_hbm, p_hbm,
          ridx, cidx, xa, ya, xb, yb, pa, pb, gsA, gsB, wsA, wsB):
        wid = lax.axis_index("c") * NS + lax.axis_index("s")
        pltpu.sync_copy(row_hbm.at[wid], ridx)
        pltpu.sync_copy(col_hbm.at[wid], cidx)

        def gather(t, xv, yv, sem):
            pltpu.async_copy(nf_hbm.at[ridx.at[t]], xv, sem)
            pltpu.async_copy(nf_hbm.at[cidx.at[t]], yv, sem)

        def gwait(xv, yv, sem):
            pltpu.make_async_copy(nf_hbm.at[ridx.at[0]], xv, sem).wait()
            pltpu.make_async_copy(nf_hbm.at[cidx.at[0]], yv, sem).wait()

        def compute(xv, yv, pv):
            def edge(e, carry):
                for q in range(8):
                    pv[e, pl.ds(16 * q, 16)] = (
                        xv[e, pl.ds(16 * q, 16)] * yv[e, pl.ds(16 * q, 16)])
                return carry

            lax.fori_loop(0, CH, edge, 0)

        def wstart(t, pv, sem):
            off = pl.multiple_of(wid * epw + t * CH, 8)
            pltpu.async_copy(pv, p_hbm.at[pl.ds(off, CH)], sem)

        def wwait(pv, sem):
            pltpu.make_async_copy(pv, p_hbm.at[pl.ds(0, CH)], sem).wait()

        gather(0, xa, ya, gsA)

        def body(t, carry):
            @pl.when(t > 0)
            def _():
                wwait(pa, wsA)

            gather(2 * t + 1, xb, yb, gsB)
            gwait(xa, ya, gsA)
            compute(xa, ya, pa)
            wstart(2 * t, pa, wsA)
            gather(2 * t + 2, xa, ya, gsA)

            @pl.when(t > 0)
            def _():
                wwait(pb, wsB)

            gwait(xb, yb, gsB)
            compute(xb, yb, pb)
            wstart(2 * t + 1, pb, wsB)
            return carry

        if NT % 2 == 1:
            lax.fori_loop(0, (NT - 1) // 2, body, 0)
            wwait(pb, wsB)
            wwait(pa, wsA)
            gwait(xa, ya, gsA)
            compute(xa, ya, pa)
            wstart(NT - 1, pa, wsA)
            wwait(pa, wsA)
        else:
            lax.fori_loop(0, NT // 2 - 1, body, 0)
            gather(NT - 1, xb, yb, gsB)
            wwait(pa, wsA)
            gwait(xa, ya, gsA)
            compute(xa, ya, pa)
            wstart(NT - 2, pa, wsA)
            wwait(pb, wsB)
            gwait(xb, yb, gsB)
            compute(xb, yb, pb)
            wstart(NT - 1, pb, wsB)
            wwait(pa, wsA)
            wwait(pb, wsB)

    return k(nf, row2d, col2d)


def _tc_messages(p, ea, wx, we, wo, eh, blk_off):
    def body(p_ref, ea_ref, wx_ref, we_ref, wo_ref, out_ref):
        z = jnp.dot(p_ref[...].astype(jnp.bfloat16), wx_ref[...],
                    preferred_element_type=jnp.float32)
        g = jnp.dot(ea_ref[...], we_ref[...],
                    preferred_element_type=jnp.float32)
        z = z * g
        z = z * (1.0 / (1.0 + jnp.exp(-z)))
        out_ref[...] = jnp.dot(z.astype(jnp.bfloat16), wo_ref[...],
                               preferred_element_type=jnp.float32)

    return pl.pallas_call(
        body,
        grid=(eh // BE,),
        in_specs=[
            pl.BlockSpec((BE, D), lambda i: (i, 0)),
            pl.BlockSpec((BE, DE), lambda i, o=blk_off: (i + o, 0)),
            pl.BlockSpec((D, DH), lambda i: (0, 0)),
            pl.BlockSpec((DE, DH), lambda i: (0, 0)),
            pl.BlockSpec((DH, DO), lambda i: (0, 0)),
        ],
        out_specs=pl.BlockSpec((BE, DO), lambda i: (i, 0)),
        out_shape=jax.ShapeDtypeStruct((eh, DO), jnp.float32),
    )(p, ea, wx, we, wo)


def _sc_scatter(messages, col2d, zeros, eh):
    epw = eh // NW
    NT = epw // CH

    @functools.partial(
        pl.kernel,
        out_type=jax.ShapeDtypeStruct((NC * N, D), jnp.float32),
        mesh=_sc_mesh(),
        scratch_types=[
            pltpu.VMEM((NT, CH), jnp.int32),
            pltpu.VMEM((CH, D), jnp.float32),
            pltpu.VMEM((CH, D), jnp.float32),
            pltpu.VMEM_SHARED((N, D), jnp.float32),
            pltpu.SemaphoreType.DMA,
            pltpu.SemaphoreType.DMA,
        ],
    )
    def k(msg_hbm, col_hbm, zero_hbm, out_hbm,
          cidx_v, ma, mb, agg_sh, lsA, lsB):
        cid = lax.axis_index("c")
        sid = lax.axis_index("s")
        wid = cid * NS + sid
        rz = 632
        zoff = pl.multiple_of(jnp.where(sid == NS - 1, N - rz, sid * rz), 8)
        pltpu.sync_copy(zero_hbm.at[pl.ds(zoff, rz)],
                        agg_sh.at[pl.ds(zoff, rz)])
        plsc.subcore_barrier()

        pltpu.sync_copy(col_hbm.at[wid], cidx_v)

        def load(t, mv, sem):
            off = pl.multiple_of(wid * epw + t * CH, 8)
            pltpu.async_copy(msg_hbm.at[pl.ds(off, CH)], mv, sem)

        def lwait(mv, sem):
            pltpu.make_async_copy(msg_hbm.at[pl.ds(0, CH)], mv, sem).wait()

        def scat(t, mv):
            pltpu.sync_copy(mv, agg_sh.at[cidx_v.at[t]], add=True)

        load(0, ma, lsA)

        def body(t, carry):
            lwait(ma, lsA)
            load(2 * t + 1, mb, lsB)
            scat(2 * t, ma)
            load(2 * t + 2, ma, lsA)
            lwait(mb, lsB)
            scat(2 * t + 1, mb)
            return carry

        if NT % 2 == 1:
            lax.fori_loop(0, (NT - 1) // 2, body, 0)
            lwait(ma, lsA)
            scat(NT - 1, ma)
        else:
            lax.fori_loop(0, NT // 2 - 1, body, 0)
            load(NT - 1, mb, lsB)
            lwait(ma, lsA)
            scat(NT - 2, ma)
            lwait(mb, lsB)
            scat(NT - 1, mb)
        plsc.subcore_barrier()
        pltpu.sync_copy(agg_sh.at[pl.ds(zoff, rz)],
                        out_hbm.at[pl.ds(pl.multiple_of(cid * N + zoff, 8), rz)])

    return k(messages, col2d, zeros)


def _tc_update(x, agg_a, agg_b, wx, wm, wo):
    BN = 1000
    nblk = N // BN

    def body(x_ref, a0_ref, a1_ref, a2_ref, a3_ref,
             wx_ref, wm_ref, wo_ref, out_ref):
        a = (a0_ref[...] + a1_ref[...]) + (a2_ref[...] + a3_ref[...])
        u = jnp.dot(x_ref[...], wx_ref[...], preferred_element_type=jnp.float32)
        u = u * jnp.dot(a, wm_ref[...], preferred_element_type=jnp.float32)
        u = u * (1.0 / (1.0 + jnp.exp(-u)))
        out_ref[...] = jnp.dot(u, wo_ref[...], preferred_element_type=jnp.float32)

    return pl.pallas_call(
        body,
        grid=(nblk,),
        in_specs=[
            pl.BlockSpec((BN, D), lambda i: (i, 0)),
            pl.BlockSpec((BN, D), lambda i: (i, 0)),
            pl.BlockSpec((BN, D), lambda i: (i + nblk, 0)),
            pl.BlockSpec((BN, D), lambda i: (i, 0)),
            pl.BlockSpec((BN, D), lambda i: (i + nblk, 0)),
            pl.BlockSpec((D, DH), lambda i: (0, 0)),
            pl.BlockSpec((DO, DH), lambda i: (0, 0)),
            pl.BlockSpec((DH, DO), lambda i: (0, 0)),
        ],
        out_specs=pl.BlockSpec((BN, DO), lambda i: (i, 0)),
        out_shape=jax.ShapeDtypeStruct((N, DO), jnp.float32),
    )(x, agg_a, agg_a, agg_b, agg_b, wx, wm, wo)


def kernel(node_features, pos, edge_index, edge_attr,
           W_msg_x, W_msg_e, W_msg_out, W_upd_x, W_upd_m, W_upd_out):
    del pos
    row = edge_index[0].astype(jnp.int32)
    col = edge_index[1].astype(jnp.int32)
    wx16 = W_msg_x.astype(jnp.bfloat16)
    wo16 = W_msg_out.astype(jnp.bfloat16)
    zeros = jnp.zeros((N, D), jnp.float32)

    aggs = []
    for off, eh in ((0, E0), (E0, E1)):
        sl = slice(off, off + eh)
        nt = eh // NW // CH
        row2d = row[sl].reshape(NW, nt, CH)
        col2d = col[sl].reshape(NW, nt, CH)
        p = _sc_gather_prod(node_features, row2d, col2d, eh)
        messages = _tc_messages(p, edge_attr, wx16, W_msg_e, wo16, eh,
                                off // BE)
        aggs.append(_sc_scatter(messages, col2d, zeros, eh))

    return _tc_update(node_features, aggs[0], aggs[1],
                      W_upd_x, W_upd_m, W_upd_out)


# speedup vs baseline: 1.5851x; 1.0599x over previous
import functools

import jax
import jax.numpy as jnp
from jax import lax
from jax.experimental import pallas as pl
from jax.experimental.pallas import tpu as pltpu
from jax.experimental.pallas import tpu_sc as plsc

N = 10000
E = 320000
D = 128
DE = 16
DH = 256
DO = 128

NC = 2
NS = 16
NW = NC * NS
E0 = 128000
E1 = E - E0
CH = 80
BE = 4000


def _sc_mesh():
    return plsc.VectorSubcoreMesh(
        core_axis_name="c", subcore_axis_name="s", num_cores=NC, num_subcores=NS
    )


def _sc_gather_prod(nf, row2d, col2d, eh):
    epw = eh // NW
    NT = epw // CH

    @functools.partial(
        pl.kernel,
        out_type=jax.ShapeDtypeStruct((eh, D), jnp.float32),
        mesh=_sc_mesh(),
        scratch_types=[
            pltpu.VMEM((NT, CH), jnp.int32),
            pltpu.VMEM((NT, CH), jnp.int32),
            pltpu.VMEM((CH, D), jnp.float32),
            pltpu.VMEM((CH, D), jnp.float32),
            pltpu.VMEM((CH, D), jnp.float32),
            pltpu.VMEM((CH, D), jnp.float32),
            pltpu.VMEM((CH, D), jnp.float32),
            pltpu.VMEM((CH, D), jnp.float32),
            pltpu.SemaphoreType.DMA,
            pltpu.SemaphoreType.DMA,
            pltpu.SemaphoreType.DMA,
            pltpu.SemaphoreType.DMA,
        ],
    )
    def k(nf_hbm, row_hbm, col_hbm, p_hbm,
          ridx, cidx, xa, ya, xb, yb, pa, pb, gsA, gsB, wsA, wsB):
        wid = lax.axis_index("c") * NS + lax.axis_index("s")
        pltpu.sync_copy(row_hbm.at[wid], ridx)
        pltpu.sync_copy(col_hbm.at[wid], cidx)

        def gather(t, xv, yv, sem):
            pltpu.async_copy(nf_hbm.at[ridx.at[t]], xv, sem)
            pltpu.async_copy(nf_hbm.at[cidx.at[t]], yv, sem)

        def gwait(xv, yv, sem):
            pltpu.make_async_copy(nf_hbm.at[ridx.at[0]], xv, sem).wait()
            pltpu.make_async_copy(nf_hbm.at[cidx.at[0]], yv, sem).wait()

        def compute(xv, yv, pv):
            def edge(e, carry):
                for q in range(8):
                    pv[e, pl.ds(16 * q, 16)] = (
                        xv[e, pl.ds(16 * q, 16)] * yv[e, pl.ds(16 * q, 16)])
                return carry

            lax.fori_loop(0, CH, edge, 0)

        def wstart(t, pv, sem):
            off = pl.multiple_of(wid * epw + t * CH, 8)
            pltpu.async_copy(pv, p_hbm.at[pl.ds(off, CH)], sem)

        def wwait(pv, sem):
            pltpu.make_async_copy(pv, p_hbm.at[pl.ds(0, CH)], sem).wait()

        gather(0, xa, ya, gsA)

        def body(t, carry):
            @pl.when(t > 0)
            def _():
                wwait(pa, wsA)

            gather(2 * t + 1, xb, yb, gsB)
            gwait(xa, ya, gsA)
            compute(xa, ya, pa)
            wstart(2 * t, pa, wsA)
            gather(2 * t + 2, xa, ya, gsA)

            @pl.when(t > 0)
            def _():
                wwait(pb, wsB)

            gwait(xb, yb, gsB)
            compute(xb, yb, pb)
            wstart(2 * t + 1, pb, wsB)
            return carry

        if NT % 2 == 1:
            lax.fori_loop(0, (NT - 1) // 2, body, 0)
            wwait(pb, wsB)
            wwait(pa, wsA)
            gwait(xa, ya, gsA)
            compute(xa, ya, pa)
            wstart(NT - 1, pa, wsA)
            wwait(pa, wsA)
        else:
            lax.fori_loop(0, NT // 2 - 1, body, 0)
            gather(NT - 1, xb, yb, gsB)
            wwait(pa, wsA)
            gwait(xa, ya, gsA)
            compute(xa, ya, pa)
            wstart(NT - 2, pa, wsA)
            wwait(pb, wsB)
            gwait(xb, yb, gsB)
            compute(xb, yb, pb)
            wstart(NT - 1, pb, wsB)
            wwait(pa, wsA)
            wwait(pb, wsB)

    return k(nf, row2d, col2d)


def _tc_messages(p, ea, wx, we, wo, eh, blk_off):
    def body(p_ref, ea_ref, wx_ref, we_ref, wo_ref, out_ref):
        z = jnp.dot(p_ref[...].astype(jnp.bfloat16), wx_ref[...],
                    preferred_element_type=jnp.float32)
        g = jnp.dot(ea_ref[...], we_ref[...],
                    preferred_element_type=jnp.float32)
        z = z * g
        z = z * (1.0 / (1.0 + jnp.exp(-z)))
        out_ref[...] = jnp.dot(z.astype(jnp.bfloat16), wo_ref[...],
                               preferred_element_type=jnp.float32)

    return pl.pallas_call(
        body,
        grid=(eh // BE,),
        in_specs=[
            pl.BlockSpec((BE, D), lambda i: (i, 0)),
            pl.BlockSpec((BE, DE), lambda i, o=blk_off: (i + o, 0)),
            pl.BlockSpec((D, DH), lambda i: (0, 0)),
            pl.BlockSpec((DE, DH), lambda i: (0, 0)),
            pl.BlockSpec((DH, DO), lambda i: (0, 0)),
        ],
        out_specs=pl.BlockSpec((BE, DO), lambda i: (i, 0)),
        out_shape=jax.ShapeDtypeStruct((eh, DO), jnp.float32),
    )(p, ea, wx, we, wo)


def _sc_scatter(messages, col2d, zeros, eh):
    epw = eh // NW
    NT = epw // CH

    @functools.partial(
        pl.kernel,
        out_type=jax.ShapeDtypeStruct((NC * N, D), jnp.float32),
        mesh=_sc_mesh(),
        scratch_types=[
            pltpu.VMEM((NT, CH), jnp.int32),
            pltpu.VMEM((CH, D), jnp.float32),
            pltpu.VMEM((CH, D), jnp.float32),
            pltpu.VMEM_SHARED((N, D), jnp.float32),
            pltpu.SemaphoreType.DMA,
            pltpu.SemaphoreType.DMA,
        ],
    )
    def k(msg_hbm, col_hbm, zero_hbm, out_hbm,
          cidx_v, ma, mb, agg_sh, lsA, lsB):
        cid = lax.axis_index("c")
        sid = lax.axis_index("s")
        wid = cid * NS + sid
        rz = 632
        zoff = pl.multiple_of(jnp.where(sid == NS - 1, N - rz, sid * rz), 8)
        pltpu.sync_copy(zero_hbm.at[pl.ds(zoff, rz)],
                        agg_sh.at[pl.ds(zoff, rz)])
        plsc.subcore_barrier()

        pltpu.sync_copy(col_hbm.at[wid], cidx_v)

        def load(t, mv, sem):
            off = pl.multiple_of(wid * epw + t * CH, 8)
            pltpu.async_copy(msg_hbm.at[pl.ds(off, CH)], mv, sem)

        def lwait(mv, sem):
            pltpu.make_async_copy(msg_hbm.at[pl.ds(0, CH)], mv, sem).wait()

        def scat(t, mv):
            pltpu.sync_copy(mv, agg_sh.at[cidx_v.at[t]], add=True)

        load(0, ma, lsA)

        def body(t, carry):
            lwait(ma, lsA)
            load(2 * t + 1, mb, lsB)
            scat(2 * t, ma)
            load(2 * t + 2, ma, lsA)
            lwait(mb, lsB)
            scat(2 * t + 1, mb)
            return carry

        if NT % 2 == 1:
            lax.fori_loop(0, (NT - 1) // 2, body, 0)
            lwait(ma, lsA)
            scat(NT - 1, ma)
        else:
            lax.fori_loop(0, NT // 2 - 1, body, 0)
            load(NT - 1, mb, lsB)
            lwait(ma, lsA)
            scat(NT - 2, ma)
            lwait(mb, lsB)
            scat(NT - 1, mb)
        plsc.subcore_barrier()
        pltpu.sync_copy(agg_sh.at[pl.ds(zoff, rz)],
                        out_hbm.at[pl.ds(pl.multiple_of(cid * N + zoff, 8), rz)])

    return k(messages, col2d, zeros)


def _tc_update(x, agg_a, agg_b, wx, wm, wo):
    BN = 1000
    nblk = N // BN

    def body(x_ref, a0_ref, a1_ref, a2_ref, a3_ref,
             wx_ref, wm_ref, wo_ref, out_ref):
        a = (a0_ref[...] + a1_ref[...]) + (a2_ref[...] + a3_ref[...])
        u = jnp.dot(x_ref[...], wx_ref[...], preferred_element_type=jnp.float32)
        u = u * jnp.dot(a, wm_ref[...], preferred_element_type=jnp.float32)
        u = u * (1.0 / (1.0 + jnp.exp(-u)))
        out_ref[...] = jnp.dot(u, wo_ref[...], preferred_element_type=jnp.float32)

    return pl.pallas_call(
        body,
        grid=(nblk,),
        in_specs=[
            pl.BlockSpec((BN, D), lambda i: (i, 0)),
            pl.BlockSpec((BN, D), lambda i: (i, 0)),
            pl.BlockSpec((BN, D), lambda i: (i + nblk, 0)),
            pl.BlockSpec((BN, D), lambda i: (i, 0)),
            pl.BlockSpec((BN, D), lambda i: (i + nblk, 0)),
            pl.BlockSpec((D, DH), lambda i: (0, 0)),
            pl.BlockSpec((DO, DH), lambda i: (0, 0)),
            pl.BlockSpec((DH, DO), lambda i: (0, 0)),
        ],
        out_specs=pl.BlockSpec((BN, DO), lambda i: (i, 0)),
        out_shape=jax.ShapeDtypeStruct((N, DO), jnp.float32),
    )(x, agg_a, agg_a, agg_b, agg_b, wx, wm, wo)


def kernel(node_features, pos, edge_index, edge_attr,
           W_msg_x, W_msg_e, W_msg_out, W_upd_x, W_upd_m, W_upd_out):
    del pos
    row = edge_index[0].astype(jnp.int32)
    col = edge_index[1].astype(jnp.int32)
    wx16 = W_msg_x.astype(jnp.bfloat16)
    wo16 = W_msg_out.astype(jnp.bfloat16)
    zeros = jnp.zeros((N, D), jnp.float32)

    aggs = []
    for off, eh in ((0, E0), (E0, E1)):
        sl = slice(off, off + eh)
        nt = eh // NW // CH
        row2d = row[sl].reshape(NW, nt, CH)
        col2d = col[sl].reshape(NW, nt, CH)
        p = _sc_gather_prod(node_features, row2d, col2d, eh)
        messages = _tc_messages(p, edge_attr, wx16, W_msg_e, wo16, eh,
                                off // BE)
        aggs.append(_sc_scatter(messages, col2d, zeros, eh))

    return _tc_update(node_features, aggs[0], aggs[1],
                      W_upd_x, W_upd_m, W_upd_out)


# speedup vs baseline: 1.6031x; 1.0114x over previous
import functools

import jax
import jax.numpy as jnp
from jax import lax
from jax.experimental import pallas as pl
from jax.experimental.pallas import tpu as pltpu
from jax.experimental.pallas import tpu_sc as plsc

N = 10000
E = 320000
D = 128
DE = 16
DH = 256
DO = 128

NC = 2
NS = 16
NW = NC * NS
E0 = 128000
E1 = E - E0
CH = 80
BE = 8000


def _sc_mesh():
    return plsc.VectorSubcoreMesh(
        core_axis_name="c", subcore_axis_name="s", num_cores=NC, num_subcores=NS
    )


def _sc_gather_prod(nf, row2d, col2d, eh):
    epw = eh // NW
    NT = epw // CH

    @functools.partial(
        pl.kernel,
        out_type=jax.ShapeDtypeStruct((eh, D), jnp.float32),
        mesh=_sc_mesh(),
        scratch_types=[
            pltpu.VMEM((NT, CH), jnp.int32),
            pltpu.VMEM((NT, CH), jnp.int32),
            pltpu.VMEM((CH, D), jnp.float32),
            pltpu.VMEM((CH, D), jnp.float32),
            pltpu.VMEM((CH, D), jnp.float32),
            pltpu.VMEM((CH, D), jnp.float32),
            pltpu.VMEM((CH, D), jnp.float32),
            pltpu.VMEM((CH, D), jnp.float32),
            pltpu.SemaphoreType.DMA,
            pltpu.SemaphoreType.DMA,
            pltpu.SemaphoreType.DMA,
            pltpu.SemaphoreType.DMA,
        ],
    )
    def k(nf_hbm, row_hbm, col_hbm, p_hbm,
          ridx, cidx, xa, ya, xb, yb, pa, pb, gsA, gsB, wsA, wsB):
        wid = lax.axis_index("c") * NS + lax.axis_index("s")
        pltpu.sync_copy(row_hbm.at[wid], ridx)
        pltpu.sync_copy(col_hbm.at[wid], cidx)

        def gather(t, xv, yv, sem):
            pltpu.async_copy(nf_hbm.at[ridx.at[t]], xv, sem)
            pltpu.async_copy(nf_hbm.at[cidx.at[t]], yv, sem)

        def gwait(xv, yv, sem):
            pltpu.make_async_copy(nf_hbm.at[ridx.at[0]], xv, sem).wait()
            pltpu.make_async_copy(nf_hbm.at[cidx.at[0]], yv, sem).wait()

        def compute(xv, yv, pv):
            def edge(e, carry):
                for q in range(8):
                    pv[e, pl.ds(16 * q, 16)] = (
                        xv[e, pl.ds(16 * q, 16)] * yv[e, pl.ds(16 * q, 16)])
                return carry

            lax.fori_loop(0, CH, edge, 0)

        def wstart(t, pv, sem):
            off = pl.multiple_of(wid * epw + t * CH, 8)
            pltpu.async_copy(pv, p_hbm.at[pl.ds(off, CH)], sem)

        def wwait(pv, sem):
            pltpu.make_async_copy(pv, p_hbm.at[pl.ds(0, CH)], sem).wait()

        gather(0, xa, ya, gsA)

        def body(t, carry):
            @pl.when(t > 0)
            def _():
                wwait(pa, wsA)

            gather(2 * t + 1, xb, yb, gsB)
            gwait(xa, ya, gsA)
            compute(xa, ya, pa)
            wstart(2 * t, pa, wsA)
            gather(2 * t + 2, xa, ya, gsA)

            @pl.when(t > 0)
            def _():
                wwait(pb, wsB)

            gwait(xb, yb, gsB)
            compute(xb, yb, pb)
            wstart(2 * t + 1, pb, wsB)
            return carry

        if NT % 2 == 1:
            lax.fori_loop(0, (NT - 1) // 2, body, 0)
            wwait(pb, wsB)
            wwait(pa, wsA)
            gwait(xa, ya, gsA)
            compute(xa, ya, pa)
            wstart(NT - 1, pa, wsA)
            wwait(pa, wsA)
        else:
            lax.fori_loop(0, NT // 2 - 1, body, 0)
            gather(NT - 1, xb, yb, gsB)
            wwait(pa, wsA)
            gwait(xa, ya, gsA)
            compute(xa, ya, pa)
            wstart(NT - 2, pa, wsA)
            wwait(pb, wsB)
            gwait(xb, yb, gsB)
            compute(xb, yb, pb)
            wstart(NT - 1, pb, wsB)
            wwait(pa, wsA)
            wwait(pb, wsB)

    return k(nf, row2d, col2d)


def _tc_messages(p, ea, wx, we, wo, eh, blk_off):
    def body(p_ref, ea_ref, wx_ref, we_ref, wo_ref, out_ref):
        z = jnp.dot(p_ref[...].astype(jnp.bfloat16), wx_ref[...],
                    preferred_element_type=jnp.float32)
        g = jnp.dot(ea_ref[...], we_ref[...],
                    preferred_element_type=jnp.float32)
        z = z * g
        z = z * (1.0 / (1.0 + jnp.exp(-z)))
        out_ref[...] = jnp.dot(z.astype(jnp.bfloat16), wo_ref[...],
                               preferred_element_type=jnp.float32)

    return pl.pallas_call(
        body,
        grid=(eh // BE,),
        in_specs=[
            pl.BlockSpec((BE, D), lambda i: (i, 0)),
            pl.BlockSpec((BE, DE), lambda i, o=blk_off: (i + o, 0)),
            pl.BlockSpec((D, DH), lambda i: (0, 0)),
            pl.BlockSpec((DE, DH), lambda i: (0, 0)),
            pl.BlockSpec((DH, DO), lambda i: (0, 0)),
        ],
        out_specs=pl.BlockSpec((BE, DO), lambda i: (i, 0)),
        out_shape=jax.ShapeDtypeStruct((eh, DO), jnp.float32),
    )(p, ea, wx, we, wo)


def _sc_scatter(messages, col2d, zeros, eh):
    epw = eh // NW
    NT = epw // CH

    @functools.partial(
        pl.kernel,
        out_type=jax.ShapeDtypeStruct((NC * N, D), jnp.float32),
        mesh=_sc_mesh(),
        scratch_types=[
            pltpu.VMEM((NT, CH), jnp.int32),
            pltpu.VMEM((CH, D), jnp.float32),
            pltpu.VMEM((CH, D), jnp.float32),
            pltpu.VMEM_SHARED((N, D), jnp.float32),
            pltpu.SemaphoreType.DMA,
            pltpu.SemaphoreType.DMA,
        ],
    )
    def k(msg_hbm, col_hbm, zero_hbm, out_hbm,
          cidx_v, ma, mb, agg_sh, lsA, lsB):
        cid = lax.axis_index("c")
        sid = lax.axis_index("s")
        wid = cid * NS + sid
        rz = 632
        zoff = pl.multiple_of(jnp.where(sid == NS - 1, N - rz, sid * rz), 8)
        pltpu.sync_copy(zero_hbm.at[pl.ds(zoff, rz)],
                        agg_sh.at[pl.ds(zoff, rz)])
        plsc.subcore_barrier()

        pltpu.sync_copy(col_hbm.at[wid], cidx_v)

        def load(t, mv, sem):
            off = pl.multiple_of(wid * epw + t * CH, 8)
            pltpu.async_copy(msg_hbm.at[pl.ds(off, CH)], mv, sem)

        def lwait(mv, sem):
            pltpu.make_async_copy(msg_hbm.at[pl.ds(0, CH)], mv, sem).wait()

        def scat(t, mv):
            pltpu.sync_copy(mv, agg_sh.at[cidx_v.at[t]], add=True)

        load(0, ma, lsA)

        def body(t, carry):
            lwait(ma, lsA)
            load(2 * t + 1, mb, lsB)
            scat(2 * t, ma)
            load(2 * t + 2, ma, lsA)
            lwait(mb, lsB)
            scat(2 * t + 1, mb)
            return carry

        if NT % 2 == 1:
            lax.fori_loop(0, (NT - 1) // 2, body, 0)
            lwait(ma, lsA)
            scat(NT - 1, ma)
        else:
            lax.fori_loop(0, NT // 2 - 1, body, 0)
            load(NT - 1, mb, lsB)
            lwait(ma, lsA)
            scat(NT - 2, ma)
            lwait(mb, lsB)
            scat(NT - 1, mb)
        plsc.subcore_barrier()
        pltpu.sync_copy(agg_sh.at[pl.ds(zoff, rz)],
                        out_hbm.at[pl.ds(pl.multiple_of(cid * N + zoff, 8), rz)])

    return k(messages, col2d, zeros)


def _tc_update(x, agg_a, agg_b, wx, wm, wo):
    BN = 1000
    nblk = N // BN

    def body(x_ref, a0_ref, a1_ref, a2_ref, a3_ref,
             wx_ref, wm_ref, wo_ref, out_ref):
        a = (a0_ref[...] + a1_ref[...]) + (a2_ref[...] + a3_ref[...])
        u = jnp.dot(x_ref[...], wx_ref[...], preferred_element_type=jnp.float32)
        u = u * jnp.dot(a, wm_ref[...], preferred_element_type=jnp.float32)
        u = u * (1.0 / (1.0 + jnp.exp(-u)))
        out_ref[...] = jnp.dot(u, wo_ref[...], preferred_element_type=jnp.float32)

    return pl.pallas_call(
        body,
        grid=(nblk,),
        in_specs=[
            pl.BlockSpec((BN, D), lambda i: (i, 0)),
            pl.BlockSpec((BN, D), lambda i: (i, 0)),
            pl.BlockSpec((BN, D), lambda i: (i + nblk, 0)),
            pl.BlockSpec((BN, D), lambda i: (i, 0)),
            pl.BlockSpec((BN, D), lambda i: (i + nblk, 0)),
            pl.BlockSpec((D, DH), lambda i: (0, 0)),
            pl.BlockSpec((DO, DH), lambda i: (0, 0)),
            pl.BlockSpec((DH, DO), lambda i: (0, 0)),
        ],
        out_specs=pl.BlockSpec((BN, DO), lambda i: (i, 0)),
        out_shape=jax.ShapeDtypeStruct((N, DO), jnp.float32),
    )(x, agg_a, agg_a, agg_b, agg_b, wx, wm, wo)


def kernel(node_features, pos, edge_index, edge_attr,
           W_msg_x, W_msg_e, W_msg_out, W_upd_x, W_upd_m, W_upd_out):
    del pos
    row = edge_index[0].astype(jnp.int32)
    col = edge_index[1].astype(jnp.int32)
    wx16 = W_msg_x.astype(jnp.bfloat16)
    wo16 = W_msg_out.astype(jnp.bfloat16)
    zeros = jnp.zeros((N, D), jnp.float32)

    aggs = []
    for off, eh in ((0, E0), (E0, E1)):
        sl = slice(off, off + eh)
        nt = eh // NW // CH
        row2d = row[sl].reshape(NW, nt, CH)
        col2d = col[sl].reshape(NW, nt, CH)
        p = _sc_gather_prod(node_features, row2d, col2d, eh)
        messages = _tc_messages(p, edge_attr, wx16, W_msg_e, wo16, eh,
                                off // BE)
        aggs.append(_sc_scatter(messages, col2d, zeros, eh))

    return _tc_update(node_features, aggs[0], aggs[1],
                      W_upd_x, W_upd_m, W_upd_out)
